# Initial kernel scaffold; baseline (speedup 1.0000x reference)
import jax
import jax.numpy as jnp
from jax.experimental import pallas as pl


def kernel(x, pos, edge_index, edge_weight, batch, W_in, b_in, Wc0, bc0, Wc1, bc1, Wc2, bc2, Wc3, bc3, g1, be1, W1, b1, g2, be2, W2, b2, g3, be3, W3, b3):
    raise NotImplementedError("write your pallas kernel here")


# speedup vs baseline: 3.8578x; 3.8578x over previous
import functools

import jax
import jax.numpy as jnp
from jax import lax
from jax.experimental import pallas as pl
from jax.experimental.pallas import tpu as pltpu
from jax.experimental.pallas import tpu_sc as plsc

N = 10000
NP = 10240
E = 320000
DF = 128
PD = 3
H = 146
F = 160
FH = 80
G = 8
NCLS = 10

NC2 = 2
NS = 16
NW = NC2 * NS
CH = 128
NCHUNK = 81
PW = NCHUNK * CH
EP = NW * PW
RPT = NP // NS

_f32 = jnp.float32
_i32 = jnp.int32

_MESH = plsc.VectorSubcoreMesh(
    core_axis_name="c", subcore_axis_name="s", num_cores=NC2, num_subcores=NS)
_SC_PARAMS = pltpu.CompilerParams(
    needs_layout_passes=False, use_tc_tiling_on_sc=False)


@functools.partial(
    pl.kernel,
    out_type=jax.ShapeDtypeStruct((NW, NP), _f32),
    mesh=_MESH,
    compiler_params=_SC_PARAMS,
    scratch_types=[
        pltpu.VMEM((PW,), _i32),
        pltpu.VMEM((PW,), _f32),
        pltpu.VMEM((NP,), _f32),
    ],
)
def _sc_deg(col_hbm, ew_hbm, out_hbm, colv, ewv, deg):
    ci = lax.axis_index("c")
    si = lax.axis_index("s")
    wid = si * NC2 + ci
    pltpu.sync_copy(col_hbm.at[wid], colv)
    pltpu.sync_copy(ew_hbm.at[wid], ewv)
    zero = jnp.zeros((16,), _f32)

    def zb(i, carry):
        deg[pl.ds(i * 16, 16)] = zero
        return carry
    lax.fori_loop(0, NP // 16, zb, 0, unroll=8)

    def acc(i, carry):
        idx = colv[pl.ds(i * 16, 16)]
        w = ewv[pl.ds(i * 16, 16)]
        plsc.addupdate_scatter(deg, [idx], w)
        return carry
    lax.fori_loop(0, PW // 16, acc, 0, unroll=4)
    pltpu.sync_copy(deg, out_hbm.at[wid])


@functools.partial(
    pl.kernel,
    out_type=jax.ShapeDtypeStruct((2, NC2, NS, RPT, FH), _f32),
    mesh=_MESH,
    compiler_params=_SC_PARAMS,
    scratch_types=[
        pltpu.VMEM((NCHUNK, CH), _i32),
        pltpu.VMEM((NCHUNK, CH), _i32),
        pltpu.VMEM((PW,), _f32),
        pltpu.VMEM((CH, FH), _f32),
        pltpu.VMEM((CH, FH), _f32),
        pltpu.VMEM((CH, FH), _f32),
        pltpu.VMEM_SHARED((NP, FH), _f32),
        pltpu.SemaphoreType.DMA,
        pltpu.SemaphoreType.DMA,
        pltpu.SemaphoreType.DMA,
        pltpu.SemaphoreType.DMA,
        pltpu.SemaphoreType.DMA,
        pltpu.SemaphoreType.DMA,
    ],
)
def _sc_edge(qa_hbm, qb_hbm, row_hbm, col_hbm, ew_hbm, out_hbm,
             rowv, colv, ewv, b0, b1, b2, acc, g0, g1, g2, s0, s1, s2):
    bufs = (b0, b1, b2)
    gsems = (g0, g1, g2)
    ssems = (s0, s1, s2)
    ci = lax.axis_index("c")
    si = lax.axis_index("s")
    wid = si * NC2 + ci
    pltpu.sync_copy(row_hbm.at[wid], rowv)
    pltpu.sync_copy(col_hbm.at[wid], colv)
    pltpu.sync_copy(ew_hbm.at[wid], ewv)

    zero = jnp.zeros((16,), _f32)

    for half, q_hbm in enumerate((qa_hbm, qb_hbm)):
        def zrow(r, carry):
            for f in range(FH // 16):
                b0[r, pl.ds(f * 16, 16)] = zero
            return carry
        lax.fori_loop(0, CH, zrow, 0)
        for z in range(RPT // CH):
            pltpu.sync_copy(b0, acc.at[pl.ds(si * RPT + z * CH, CH)])
        plsc.subcore_barrier()

        def g_start(cc, b, sem):
            pltpu.async_copy(q_hbm.at[rowv.at[cc]], b, sem)

        def g_wait(cc, b, sem):
            pltpu.make_async_copy(q_hbm.at[rowv.at[cc]], b, sem).wait()

        def s_start(cc, b, sem):
            pltpu.async_copy(b, acc.at[colv.at[cc]], sem, add=True)

        def s_wait(cc, b, sem):
            pltpu.make_async_copy(b, acc.at[colv.at[cc]], sem).wait()

        def scale(cc, b):
            base = cc * CH

            def grp(g_, carry):
                r0 = g_ * 16
                for j in range(16):
                    r = r0 + j
                    bc = plsc.load_gather(
                        ewv, [jnp.full((16,), base + r, _i32)])
                    for f in range(FH // 16):
                        b[r, pl.ds(f * 16, 16)] = b[r, pl.ds(f * 16, 16)] * bc
                return carry
            lax.fori_loop(0, CH // 16, grp, 0)

        g_start(0, bufs[0], gsems[0])

        def outer(o, carry):
            for b in range(3):
                j = o * 3 + b
                nb = (b + 1) % 3

                @pl.when(j + 1 < NCHUNK)
                def _():
                    @pl.when(j >= 2)
                    def _():
                        s_wait(j - 2, bufs[nb], ssems[nb])
                    g_start(j + 1, bufs[nb], gsems[nb])

                g_wait(j, bufs[b], gsems[b])
                scale(j, bufs[b])
                s_start(j, bufs[b], ssems[b])
            return carry
        lax.fori_loop(0, NCHUNK // 3, outer, 0)
        s_wait(NCHUNK - 2, bufs[(NCHUNK - 2) % 3], ssems[(NCHUNK - 2) % 3])
        s_wait(NCHUNK - 1, bufs[(NCHUNK - 1) % 3], ssems[(NCHUNK - 1) % 3])
        plsc.subcore_barrier()
        pltpu.sync_copy(acc.at[pl.ds(si * RPT, RPT)], out_hbm.at[half, ci, si])


BM = 2048


def _mxu(a, b):
    return jnp.dot(a.astype(jnp.bfloat16), b.astype(jnp.bfloat16),
                   preferred_element_type=_f32)

def _relu(x):
    return jnp.maximum(x, 0.0)


def _dis_e(deg):
    return jnp.where(deg > 0.0, lax.rsqrt(jnp.maximum(deg, 1e-30)), 0.0)


def _split(q, qa, qb):
    qa[...] = q[:, :FH]
    qb[...] = q[:, FH:]


def _tc_in_body(xp, winp, binp, wc0, degp, qa, qb, s0):
    deg = jnp.sum(degp[...], axis=0)
    dis0 = lax.rsqrt(deg + 1.0)
    h = _mxu(xp[...], winp[...]) + binp[...]
    hw = _mxu(h, wc0[...])
    _split(hw * dis0[:, None], qa, qb)
    s0[...] = hw * (dis0 * dis0)[:, None]


_tc_in = pl.pallas_call(
    _tc_in_body,
    grid=(NP // BM,),
    in_specs=[
        pl.BlockSpec((BM, F), lambda i: (i, 0)),
        pl.BlockSpec((F, F), lambda i: (0, 0)),
        pl.BlockSpec((1, F), lambda i: (0, 0)),
        pl.BlockSpec((F, F), lambda i: (0, 0)),
        pl.BlockSpec((NW, BM), lambda i: (0, i)),
    ],
    out_specs=[
        pl.BlockSpec((BM, FH), lambda i: (i, 0)),
        pl.BlockSpec((BM, FH), lambda i: (i, 0)),
        pl.BlockSpec((BM, F), lambda i: (i, 0)),
    ],
    out_shape=[
        jax.ShapeDtypeStruct((NP, FH), _f32),
        jax.ShapeDtypeStruct((NP, FH), _f32),
        jax.ShapeDtypeStruct((NP, F), _f32),
    ],
)


def _acc_sum(accl, accr):
    return jnp.concatenate([accl[0] + accl[1], accr[0] + accr[1]], axis=-1)


def _tc_mid1_body(accl, accr, degp, s0, b0, wc, qa, qb):
    deg = jnp.sum(degp[...], axis=0)
    dis0 = lax.rsqrt(deg + 1.0)
    dise = _dis_e(deg)
    acs = _acc_sum(accl, accr)
    h = _relu(dis0[:, None] * acs + s0[...] + b0[...])
    q = _mxu(h, wc[...]) * dise[:, None]
    _split(q, qa, qb)


def _tc_mid_body(accl, accr, degp, bprev, wc, qa, qb):
    deg = jnp.sum(degp[...], axis=0)
    dise = _dis_e(deg)
    acs = _acc_sum(accl, accr)
    h = _relu(dise[:, None] * acs + bprev[...])
    q = _mxu(h, wc[...]) * dise[:, None]
    _split(q, qa, qb)


_ACC_SPEC = pl.BlockSpec((NC2, BM, FH), lambda i: (0, i, 0))
_QOUT = dict(
    out_specs=[
        pl.BlockSpec((BM, FH), lambda i: (i, 0)),
        pl.BlockSpec((BM, FH), lambda i: (i, 0)),
    ],
    out_shape=[
        jax.ShapeDtypeStruct((NP, FH), _f32),
        jax.ShapeDtypeStruct((NP, FH), _f32),
    ],
)

_tc_mid1 = pl.pallas_call(
    _tc_mid1_body,
    grid=(NP // BM,),
    in_specs=[
        _ACC_SPEC,
        _ACC_SPEC,
        pl.BlockSpec((NW, BM), lambda i: (0, i)),
        pl.BlockSpec((BM, F), lambda i: (i, 0)),
        pl.BlockSpec((1, F), lambda i: (0, 0)),
        pl.BlockSpec((F, F), lambda i: (0, 0)),
    ],
    **_QOUT,
)

_tc_mid = pl.pallas_call(
    _tc_mid_body,
    grid=(NP // BM,),
    in_specs=[
        _ACC_SPEC,
        _ACC_SPEC,
        pl.BlockSpec((NW, BM), lambda i: (0, i)),
        pl.BlockSpec((1, F), lambda i: (0, 0)),
        pl.BlockSpec((F, F), lambda i: (0, 0)),
    ],
    **_QOUT,
)


def _bn(x, g, b, eps=1e-5):
    m = jnp.mean(x, axis=0, keepdims=True)
    v = jnp.mean((x - m) * (x - m), axis=0, keepdims=True)
    return (x - m) / jnp.sqrt(v + eps) * g + b


def _tc_head_body(accl, accr, degp, oh, b3, g1, be1, w1, b1, g2, be2, w2, b2,
                  g3, be3, w3, b3h, out):
    deg = jnp.sum(degp[...], axis=0)
    dise = _dis_e(deg)
    acs = jnp.concatenate([accl[0] + accl[1], accr[0] + accr[1]], axis=-1)
    h4 = dise[:, None] * acs + b3[...]
    ohv = oh[...]
    psum = jax.lax.dot_general(ohv, h4, (((1,), (0,)), ((), ())),
                           precision=jax.lax.Precision.HIGHEST,
                           preferred_element_type=_f32)
    cnt = jnp.sum(ohv, axis=1, keepdims=True)
    pooled = psum / jnp.maximum(cnt, 1.0)
    o = _mxu(_relu(_bn(pooled, g1[...], be1[...])), w1[...]) + b1[...]
    o = _mxu(_relu(_bn(o, g2[...], be2[...])), w2[...]) + b2[...]
    o = _mxu(_relu(_bn(o, g3[...], be3[...])), w3[...]) + b3h[...]
    out[...] = o


_tc_head = pl.pallas_call(
    _tc_head_body,
    out_shape=jax.ShapeDtypeStruct((G, 128), _f32),
)


def kernel(x, pos, edge_index, edge_weight, batch, W_in, b_in,
           Wc0, bc0, Wc1, bc1, Wc2, bc2, Wc3, bc3,
           g1, be1, W1, b1, g2, be2, W2, b2, g3, be3, W3, b3):
    row = edge_index[0]
    col = edge_index[1]
    zpad_i = jnp.zeros((EP - E,), _i32)
    zpad_f = jnp.zeros((EP - E,), _f32)
    row_w = jnp.concatenate([row, zpad_i]).reshape(NW, NCHUNK, CH)
    col_p = jnp.concatenate([col, zpad_i])
    col_w = col_p.reshape(NW, NCHUNK, CH)
    col_d = col_p.reshape(NW, PW)
    ew_w = jnp.concatenate([edge_weight, zpad_f]).reshape(NW, PW)

    degp = _sc_deg(col_d, ew_w)

    xp = jnp.zeros((NP, F), _f32)
    xp = xp.at[:N, :DF].set(x).at[:N, DF:DF + PD].set(pos)
    winp = jnp.zeros((F, F), _f32).at[:DF + PD, :H].set(W_in)
    binp = jnp.zeros((1, F), _f32).at[0, :H].set(b_in)

    def padw(w):
        return jnp.zeros((F, F), _f32).at[:H, :H].set(w)

    def padb(b):
        return jnp.zeros((1, F), _f32).at[0, :H].set(b)

    wc = [padw(Wc0), padw(Wc1), padw(Wc2), padw(Wc3)]
    bc = [padb(bc0), padb(bc1), padb(bc2), padb(bc3)]

    def edge_pass(qa, qb):
        out = _sc_edge(qa, qb, row_w, col_w, ew_w)
        accl = out[0].reshape(NC2, NP, FH)
        accr = out[1].reshape(NC2, NP, FH)
        return accl, accr

    q0a, q0b, s0 = _tc_in(xp, winp, binp, wc[0], degp)
    accl, accr = edge_pass(q0a, q0b)
    q1a, q1b = _tc_mid1(accl, accr, degp, s0, bc[0], wc[1])
    accl, accr = edge_pass(q1a, q1b)
    q2a, q2b = _tc_mid(accl, accr, degp, bc[1], wc[2])
    accl, accr = edge_pass(q2a, q2b)
    q3a, q3b = _tc_mid(accl, accr, degp, bc[2], wc[3])
    accl, accr = edge_pass(q3a, q3b)

    batch_p = jnp.concatenate([batch, jnp.full((NP - N,), 127, batch.dtype)])
    oh = (batch_p[None, :] == jnp.arange(G, dtype=batch.dtype)[:, None])
    oh = oh.astype(_f32)

    H2, H4 = H // 2, H // 4
    g1p = jnp.zeros((1, F), _f32).at[0, :H].set(g1)
    be1p = jnp.zeros((1, F), _f32).at[0, :H].set(be1)
    w1p = jnp.zeros((F, 80), _f32).at[:H, :H2].set(W1)
    b1p = jnp.zeros((1, 80), _f32).at[0, :H2].set(b1)
    g2p = jnp.zeros((1, 80), _f32).at[0, :H2].set(g2)
    be2p = jnp.zeros((1, 80), _f32).at[0, :H2].set(be2)
    w2p = jnp.zeros((80, 48), _f32).at[:H2, :H4].set(W2)
    b2p = jnp.zeros((1, 48), _f32).at[0, :H4].set(b2)
    g3p = jnp.zeros((1, 48), _f32).at[0, :H4].set(g3)
    be3p = jnp.zeros((1, 48), _f32).at[0, :H4].set(be3)
    w3p = jnp.zeros((48, 128), _f32).at[:H4, :NCLS].set(W3)
    b3p = jnp.zeros((1, 128), _f32).at[0, :NCLS].set(b3)

    o = _tc_head(accl, accr, degp, oh, bc[3], g1p, be1p, w1p, b1p,
                 g2p, be2p, w2p, b2p, g3p, be3p, w3p, b3p)
    return o[:, :NCLS]


# speedup vs baseline: 5.9790x; 1.5498x over previous
import functools

import jax
import jax.numpy as jnp
from jax import lax
from jax.experimental import pallas as pl
from jax.experimental.pallas import tpu as pltpu
from jax.experimental.pallas import tpu_sc as plsc

N = 10000
NP = 10240
E = 320000
DF = 128
PD = 3
H = 146
F = 256
FH = 128
G = 8
NCLS = 10

NC2 = 2
NS = 16
NW = NC2 * NS
CH = 120
NCHUNK = 84
PW = NCHUNK * CH
EP = NW * PW
RPT = NP // NS

_f32 = jnp.float32
_i32 = jnp.int32

_MESH = plsc.VectorSubcoreMesh(
    core_axis_name="c", subcore_axis_name="s", num_cores=NC2, num_subcores=NS)
_SC_PARAMS = pltpu.CompilerParams(
    needs_layout_passes=False, use_tc_tiling_on_sc=False)


@functools.partial(
    pl.kernel,
    out_type=jax.ShapeDtypeStruct((NW * NP,), _f32),
    mesh=_MESH,
    compiler_params=_SC_PARAMS,
    scratch_types=[
        pltpu.VMEM((PW,), _i32),
        pltpu.VMEM((PW,), _f32),
        pltpu.VMEM((NP,), _f32),
    ],
)
def _sc_deg(col_hbm, ew_hbm, out_hbm, colv, ewv, deg):
    ci = lax.axis_index("c")
    si = lax.axis_index("s")
    wid = si * NC2 + ci
    pltpu.sync_copy(col_hbm.at[pl.ds(wid * PW, PW)], colv)
    pltpu.sync_copy(ew_hbm.at[pl.ds(wid * PW, PW)], ewv)
    zero = jnp.zeros((16,), _f32)

    def zb(i, carry):
        deg[pl.ds(i * 16, 16)] = zero
        return carry
    lax.fori_loop(0, NP // 16, zb, 0, unroll=8)

    def acc(i, carry):
        idx = colv[pl.ds(i * 16, 16)]
        w = ewv[pl.ds(i * 16, 16)]
        plsc.addupdate_scatter(deg, [idx], w)
        return carry
    lax.fori_loop(0, PW // 16, acc, 0, unroll=4)
    pltpu.sync_copy(deg, out_hbm.at[pl.ds(wid * NP, NP)])


@functools.partial(
    pl.kernel,
    out_type=jax.ShapeDtypeStruct((2, NC2, NS, RPT, FH), _f32),
    mesh=_MESH,
    compiler_params=_SC_PARAMS,
    scratch_types=[
        pltpu.VMEM((CH, FH), _f32),
        pltpu.VMEM((CH, FH), _f32),
        pltpu.VMEM((CH, FH), _f32),
        pltpu.VMEM((CH,), _i32),
        pltpu.VMEM((CH,), _i32),
        pltpu.VMEM((CH,), _i32),
        pltpu.VMEM((CH,), _i32),
        pltpu.VMEM((CH,), _i32),
        pltpu.VMEM((CH,), _i32),
        pltpu.VMEM((CH,), _f32),
        pltpu.VMEM((CH,), _f32),
        pltpu.VMEM((CH,), _f32),
        pltpu.VMEM_SHARED((NP, FH), _f32),
        pltpu.SemaphoreType.DMA,
        pltpu.SemaphoreType.DMA,
        pltpu.SemaphoreType.DMA,
        pltpu.SemaphoreType.DMA,
        pltpu.SemaphoreType.DMA,
        pltpu.SemaphoreType.DMA,
        pltpu.SemaphoreType.DMA,
        pltpu.SemaphoreType.DMA,
        pltpu.SemaphoreType.DMA,
    ],
)
def _sc_edge(qa_hbm, qb_hbm, row_hbm, col_hbm, ew_hbm, out_hbm,
             b0, b1, b2, r0, r1, r2, c0, c1, c2, w0, w1, w2, acc,
             g0, g1, g2, s0, s1, s2, i0, i1, i2):
    bufs = (b0, b1, b2)
    rows = (r0, r1, r2)
    cols = (c0, c1, c2)
    ews = (w0, w1, w2)
    gsems = (g0, g1, g2)
    ssems = (s0, s1, s2)
    isems = (i0, i1, i2)
    ci = lax.axis_index("c")
    si = lax.axis_index("s")
    wid = si * NC2 + ci
    ebase = wid * PW

    zero = jnp.zeros((16,), _f32)

    def idx_start(cc, sl):
        off = ebase + cc * CH
        pltpu.async_copy(row_hbm.at[pl.ds(off, CH)], rows[sl], isems[sl])
        pltpu.async_copy(col_hbm.at[pl.ds(off, CH)], cols[sl], isems[sl])
        pltpu.async_copy(ew_hbm.at[pl.ds(off, CH)], ews[sl], isems[sl])

    def idx_wait(cc, sl):
        off = ebase + cc * CH
        pltpu.make_async_copy(row_hbm.at[pl.ds(off, CH)], rows[sl], isems[sl]).wait()
        pltpu.make_async_copy(col_hbm.at[pl.ds(off, CH)], cols[sl], isems[sl]).wait()
        pltpu.make_async_copy(ew_hbm.at[pl.ds(off, CH)], ews[sl], isems[sl]).wait()

    for half, q_hbm in enumerate((qa_hbm, qb_hbm)):
        def zrow(r, carry):
            for f in range(FH // 16):
                b0[r, pl.ds(f * 16, 16)] = zero
            return carry
        lax.fori_loop(0, 80, zrow, 0)
        for z in range(RPT // 80):
            pltpu.sync_copy(b0.at[pl.ds(0, 80)],
                            acc.at[pl.ds(si * RPT + z * 80, 80)])
        plsc.subcore_barrier()

        def g_start(sl):
            pltpu.async_copy(q_hbm.at[rows[sl]], bufs[sl], gsems[sl])

        def g_wait(sl):
            pltpu.make_async_copy(q_hbm.at[rows[sl]], bufs[sl], gsems[sl]).wait()

        def s_start(sl):
            pltpu.async_copy(bufs[sl], acc.at[cols[sl]], ssems[sl], add=True)

        def s_wait(sl):
            pltpu.make_async_copy(bufs[sl], acc.at[cols[sl]], ssems[sl]).wait()

        def scale(sl):
            b = bufs[sl]
            ew = ews[sl]

            def grp(g_, carry):
                q0 = g_ * 8
                for j in range(8):
                    r = q0 + j
                    bc = plsc.load_gather(ew, [jnp.full((16,), r, _i32)])
                    for f in range(FH // 16):
                        b[r, pl.ds(f * 16, 16)] = b[r, pl.ds(f * 16, 16)] * bc
                return carry
            lax.fori_loop(0, CH // 8, grp, 0)

        idx_start(0, 0)
        idx_wait(0, 0)
        g_start(0)
        idx_start(1, 1)

        def body(c, carry):
            for sl in range(3):
                cc = c * 3 + sl
                ns = (sl + 1) % 3
                nns = (sl + 2) % 3
                g_wait(sl)
                scale(sl)
                s_start(sl)

                @pl.when(cc + 2 < NCHUNK)
                def _():
                    @pl.when(cc >= 1)
                    def _():
                        s_wait(nns)
                    idx_start(cc + 2, nns)

                @pl.when(cc + 1 < NCHUNK)
                def _():
                    idx_wait(cc + 1, ns)
                    g_start(ns)
            return carry
        lax.fori_loop(0, NCHUNK // 3, body, 0)
        s_wait((NCHUNK - 3) % 3)
        s_wait((NCHUNK - 2) % 3)
        s_wait((NCHUNK - 1) % 3)
        plsc.subcore_barrier()
        pltpu.sync_copy(acc.at[pl.ds(si * RPT, RPT)], out_hbm.at[half, ci, si])


BM = 2048


def _mxu(a, b):
    return jnp.dot(a.astype(jnp.bfloat16), b.astype(jnp.bfloat16),
                   preferred_element_type=_f32)


def _relu(x):
    return jnp.maximum(x, 0.0)


def _dis_e(deg):
    return jnp.where(deg > 0.0, lax.rsqrt(jnp.maximum(deg, 1e-30)), 0.0)


def _split(q, qa, qb):
    qa[...] = q[:, :FH]
    qb[...] = q[:, FH:]


def _tc_in_body(xp, winp, binp, wc0, degp, qa, qb, sf):
    deg = jnp.sum(degp[...], axis=0)
    dis0 = lax.rsqrt(deg + 1.0)
    h = _mxu(xp[...], winp[...]) + binp[...]
    hw = _mxu(h, wc0[...])
    _split(hw * dis0[:, None], qa, qb)
    sf[...] = hw * (dis0 * dis0)[:, None]


_tc_in = pl.pallas_call(
    _tc_in_body,
    grid=(NP // BM,),
    in_specs=[
        pl.BlockSpec((BM, F), lambda i: (i, 0)),
        pl.BlockSpec((F, F), lambda i: (0, 0)),
        pl.BlockSpec((1, F), lambda i: (0, 0)),
        pl.BlockSpec((F, F), lambda i: (0, 0)),
        pl.BlockSpec((NW, BM), lambda i: (0, i)),
    ],
    out_specs=[
        pl.BlockSpec((BM, FH), lambda i: (i, 0)),
        pl.BlockSpec((BM, FH), lambda i: (i, 0)),
        pl.BlockSpec((BM, F), lambda i: (i, 0)),
    ],
    out_shape=[
        jax.ShapeDtypeStruct((NP, FH), _f32),
        jax.ShapeDtypeStruct((NP, FH), _f32),
        jax.ShapeDtypeStruct((NP, F), _f32),
    ],
)


def _acc_sum(accl, accr):
    return jnp.concatenate([accl[0] + accl[1], accr[0] + accr[1]], axis=-1)


def _tc_mid1_body(accl, accr, degp, sf, b0, wc, qa, qb):
    deg = jnp.sum(degp[...], axis=0)
    dis0 = lax.rsqrt(deg + 1.0)
    dise = _dis_e(deg)
    acs = _acc_sum(accl, accr)
    h = _relu(dis0[:, None] * acs + sf[...] + b0[...])
    q = _mxu(h, wc[...]) * dise[:, None]
    _split(q, qa, qb)


def _tc_mid_body(accl, accr, degp, bprev, wc, qa, qb):
    deg = jnp.sum(degp[...], axis=0)
    dise = _dis_e(deg)
    acs = _acc_sum(accl, accr)
    h = _relu(dise[:, None] * acs + bprev[...])
    q = _mxu(h, wc[...]) * dise[:, None]
    _split(q, qa, qb)


_ACC_SPEC = pl.BlockSpec((NC2, BM, FH), lambda i: (0, i, 0))
_QOUT = dict(
    out_specs=[
        pl.BlockSpec((BM, FH), lambda i: (i, 0)),
        pl.BlockSpec((BM, FH), lambda i: (i, 0)),
    ],
    out_shape=[
        jax.ShapeDtypeStruct((NP, FH), _f32),
        jax.ShapeDtypeStruct((NP, FH), _f32),
    ],
)

_tc_mid1 = pl.pallas_call(
    _tc_mid1_body,
    grid=(NP // BM,),
    in_specs=[
        _ACC_SPEC,
        _ACC_SPEC,
        pl.BlockSpec((NW, BM), lambda i: (0, i)),
        pl.BlockSpec((BM, F), lambda i: (i, 0)),
        pl.BlockSpec((1, F), lambda i: (0, 0)),
        pl.BlockSpec((F, F), lambda i: (0, 0)),
    ],
    **_QOUT,
)

_tc_mid = pl.pallas_call(
    _tc_mid_body,
    grid=(NP // BM,),
    in_specs=[
        _ACC_SPEC,
        _ACC_SPEC,
        pl.BlockSpec((NW, BM), lambda i: (0, i)),
        pl.BlockSpec((1, F), lambda i: (0, 0)),
        pl.BlockSpec((F, F), lambda i: (0, 0)),
    ],
    **_QOUT,
)


def _bn(x, g, b, eps=1e-5):
    m = jnp.mean(x, axis=0, keepdims=True)
    v = jnp.mean((x - m) * (x - m), axis=0, keepdims=True)
    return (x - m) / jnp.sqrt(v + eps) * g + b


def _tc_head_body(accl, accr, degp, oh, b3, g1, be1, w1, b1, g2, be2, w2, b2,
                  g3, be3, w3, b3h, out):
    deg = jnp.sum(degp[...], axis=0)
    dise = _dis_e(deg)
    acs = jnp.concatenate([accl[0] + accl[1], accr[0] + accr[1]], axis=-1)
    h4 = dise[:, None] * acs + b3[...]
    ohv = oh[...]
    psum = jax.lax.dot_general(ohv, h4, (((1,), (0,)), ((), ())),
                               precision=jax.lax.Precision.HIGHEST,
                               preferred_element_type=_f32)
    cnt = jnp.sum(ohv, axis=1, keepdims=True)
    pooled = psum / jnp.maximum(cnt, 1.0)
    o = _mxu(_relu(_bn(pooled, g1[...], be1[...])), w1[...]) + b1[...]
    o = _mxu(_relu(_bn(o, g2[...], be2[...])), w2[...]) + b2[...]
    o = _mxu(_relu(_bn(o, g3[...], be3[...])), w3[...]) + b3h[...]
    out[...] = o


_tc_head = pl.pallas_call(
    _tc_head_body,
    out_shape=jax.ShapeDtypeStruct((G, 128), _f32),
)


def kernel(x, pos, edge_index, edge_weight, batch, W_in, b_in,
           Wc0, bc0, Wc1, bc1, Wc2, bc2, Wc3, bc3,
           g1, be1, W1, b1, g2, be2, W2, b2, g3, be3, W3, b3):
    row = edge_index[0]
    col = edge_index[1]
    zpad_i = jnp.zeros((EP - E,), _i32)
    zpad_f = jnp.zeros((EP - E,), _f32)
    row_p = jnp.concatenate([row, zpad_i])
    col_p = jnp.concatenate([col, zpad_i])
    ew_p = jnp.concatenate([edge_weight, zpad_f])

    degp = _sc_deg(col_p, ew_p).reshape(NW, NP)

    xp = jnp.zeros((NP, F), _f32)
    xp = xp.at[:N, :DF].set(x).at[:N, DF:DF + PD].set(pos)
    winp = jnp.zeros((F, F), _f32).at[:DF + PD, :H].set(W_in)
    binp = jnp.zeros((1, F), _f32).at[0, :H].set(b_in)

    def padw(w):
        return jnp.zeros((F, F), _f32).at[:H, :H].set(w)

    def padb(b):
        return jnp.zeros((1, F), _f32).at[0, :H].set(b)

    wc = [padw(Wc0), padw(Wc1), padw(Wc2), padw(Wc3)]
    bc = [padb(bc0), padb(bc1), padb(bc2), padb(bc3)]

    def edge_pass(qa, qb):
        out = _sc_edge(qa, qb, row_p, col_p, ew_p)
        accl = out[0].reshape(NC2, NP, FH)
        accr = out[1].reshape(NC2, NP, FH)
        return accl, accr

    q0a, q0b, sf = _tc_in(xp, winp, binp, wc[0], degp)
    accl, accr = edge_pass(q0a, q0b)
    q1a, q1b = _tc_mid1(accl, accr, degp, sf, bc[0], wc[1])
    accl, accr = edge_pass(q1a, q1b)
    q2a, q2b = _tc_mid(accl, accr, degp, bc[1], wc[2])
    accl, accr = edge_pass(q2a, q2b)
    q3a, q3b = _tc_mid(accl, accr, degp, bc[2], wc[3])
    accl, accr = edge_pass(q3a, q3b)

    batch_p = jnp.concatenate([batch, jnp.full((NP - N,), 127, batch.dtype)])
    oh = (batch_p[None, :] == jnp.arange(G, dtype=batch.dtype)[:, None])
    oh = oh.astype(_f32)

    H2, H4 = H // 2, H // 4
    g1p = jnp.zeros((1, F), _f32).at[0, :H].set(g1)
    be1p = jnp.zeros((1, F), _f32).at[0, :H].set(be1)
    w1p = jnp.zeros((F, 80), _f32).at[:H, :H2].set(W1)
    b1p = jnp.zeros((1, 80), _f32).at[0, :H2].set(b1)
    g2p = jnp.zeros((1, 80), _f32).at[0, :H2].set(g2)
    be2p = jnp.zeros((1, 80), _f32).at[0, :H2].set(be2)
    w2p = jnp.zeros((80, 48), _f32).at[:H2, :H4].set(W2)
    b2p = jnp.zeros((1, 48), _f32).at[0, :H4].set(b2)
    g3p = jnp.zeros((1, 48), _f32).at[0, :H4].set(g3)
    be3p = jnp.zeros((1, 48), _f32).at[0, :H4].set(be3)
    w3p = jnp.zeros((48, 128), _f32).at[:H4, :NCLS].set(W3)
    b3p = jnp.zeros((1, 128), _f32).at[0, :NCLS].set(b3)

    o = _tc_head(accl, accr, degp, oh, bc[3], g1p, be1p, w1p, b1p,
                 g2p, be2p, w2p, b2p, g3p, be3p, w3p, b3p)
    return o[:, :NCLS]


# speedup vs baseline: 6.2794x; 1.0502x over previous
import functools

import jax
import jax.numpy as jnp
from jax import lax
from jax.experimental import pallas as pl
from jax.experimental.pallas import tpu as pltpu
from jax.experimental.pallas import tpu_sc as plsc

N = 10000
NP = 10240
E = 320000
DF = 128
PD = 3
H = 146
F = 256
FH = 128
G = 8
NCLS = 10

NC2 = 2
NS = 16
NW = NC2 * NS
CH = 120
NCHUNK = 84
PW = NCHUNK * CH
EP = NW * PW
RPT = NP // NS

_f32 = jnp.float32
_i32 = jnp.int32

_MESH = plsc.VectorSubcoreMesh(
    core_axis_name="c", subcore_axis_name="s", num_cores=NC2, num_subcores=NS)
_SC_PARAMS = pltpu.CompilerParams(
    needs_layout_passes=False, use_tc_tiling_on_sc=False)


@functools.partial(
    pl.kernel,
    out_type=jax.ShapeDtypeStruct((NW * NP,), _f32),
    mesh=_MESH,
    compiler_params=_SC_PARAMS,
    scratch_types=[
        pltpu.VMEM((PW,), _i32),
        pltpu.VMEM((PW,), _f32),
        pltpu.VMEM((NP,), _f32),
    ],
)
def _sc_deg(col_hbm, ew_hbm, out_hbm, colv, ewv, deg):
    ci = lax.axis_index("c")
    si = lax.axis_index("s")
    wid = si * NC2 + ci
    pltpu.sync_copy(col_hbm.at[pl.ds(wid * PW, PW)], colv)
    pltpu.sync_copy(ew_hbm.at[pl.ds(wid * PW, PW)], ewv)
    zero = jnp.zeros((16,), _f32)

    def zb(i, carry):
        deg[pl.ds(i * 16, 16)] = zero
        return carry
    lax.fori_loop(0, NP // 16, zb, 0, unroll=8)

    def acc(i, carry):
        idx = colv[pl.ds(i * 16, 16)]
        w = ewv[pl.ds(i * 16, 16)]
        plsc.addupdate_scatter(deg, [idx], w)
        return carry
    lax.fori_loop(0, PW // 16, acc, 0, unroll=4)
    pltpu.sync_copy(deg, out_hbm.at[pl.ds(wid * NP, NP)])


@functools.partial(
    pl.kernel,
    out_type=jax.ShapeDtypeStruct((2, NC2, NS, RPT, FH), _f32),
    mesh=_MESH,
    compiler_params=_SC_PARAMS,
    scratch_types=[
        pltpu.VMEM((CH, FH), _f32),
        pltpu.VMEM((CH, FH), _f32),
        pltpu.VMEM((CH, FH), _f32),
        pltpu.VMEM((CH,), _i32),
        pltpu.VMEM((CH,), _i32),
        pltpu.VMEM((CH,), _i32),
        pltpu.VMEM((CH,), _i32),
        pltpu.VMEM((CH,), _i32),
        pltpu.VMEM((CH,), _i32),
        pltpu.VMEM((CH,), _f32),
        pltpu.VMEM((CH,), _f32),
        pltpu.VMEM((CH,), _f32),
        pltpu.VMEM_SHARED((NP, FH), _f32),
        pltpu.SemaphoreType.DMA,
        pltpu.SemaphoreType.DMA,
        pltpu.SemaphoreType.DMA,
        pltpu.SemaphoreType.DMA,
        pltpu.SemaphoreType.DMA,
        pltpu.SemaphoreType.DMA,
        pltpu.SemaphoreType.DMA,
        pltpu.SemaphoreType.DMA,
        pltpu.SemaphoreType.DMA,
    ],
)
def _sc_edge(qa_hbm, qb_hbm, row_hbm, col_hbm, ew_hbm, out_hbm,
             b0, b1, b2, r0, r1, r2, c0, c1, c2, w0, w1, w2, acc,
             g0, g1, g2, s0, s1, s2, i0, i1, i2):
    bufs = (b0, b1, b2)
    rows = (r0, r1, r2)
    cols = (c0, c1, c2)
    ews = (w0, w1, w2)
    gsems = (g0, g1, g2)
    ssems = (s0, s1, s2)
    isems = (i0, i1, i2)
    ci = lax.axis_index("c")
    si = lax.axis_index("s")
    wid = si * NC2 + ci
    ebase = wid * PW

    zero = jnp.zeros((16,), _f32)

    def idx_start(cc, sl):
        off = ebase + cc * CH
        pltpu.async_copy(row_hbm.at[pl.ds(off, CH)], rows[sl], isems[sl])
        pltpu.async_copy(col_hbm.at[pl.ds(off, CH)], cols[sl], isems[sl])
        pltpu.async_copy(ew_hbm.at[pl.ds(off, CH)], ews[sl], isems[sl])

    def idx_wait(cc, sl):
        off = ebase + cc * CH
        pltpu.make_async_copy(row_hbm.at[pl.ds(off, CH)], rows[sl], isems[sl]).wait()
        pltpu.make_async_copy(col_hbm.at[pl.ds(off, CH)], cols[sl], isems[sl]).wait()
        pltpu.make_async_copy(ew_hbm.at[pl.ds(off, CH)], ews[sl], isems[sl]).wait()

    for half, q_hbm in enumerate((qa_hbm, qb_hbm)):
        def zrow(r, carry):
            for f in range(FH // 16):
                b0[r, pl.ds(f * 16, 16)] = zero
            return carry
        lax.fori_loop(0, 80, zrow, 0)
        for z in range(RPT // 80):
            pltpu.sync_copy(b0.at[pl.ds(0, 80)],
                            acc.at[pl.ds(si * RPT + z * 80, 80)])
        plsc.subcore_barrier()

        def g_start(sl):
            pltpu.async_copy(q_hbm.at[rows[sl]], bufs[sl], gsems[sl])

        def g_wait(sl):
            pltpu.make_async_copy(q_hbm.at[rows[sl]], bufs[sl], gsems[sl]).wait()

        def s_start(sl):
            pltpu.async_copy(bufs[sl], acc.at[cols[sl]], ssems[sl], add=True)

        def s_wait(sl):
            pltpu.make_async_copy(bufs[sl], acc.at[cols[sl]], ssems[sl]).wait()

        nf = (FH // 16) if half == 0 else 2

        def scale(sl):
            b = bufs[sl]
            ew = ews[sl]

            def grp(g_, carry):
                q0 = g_ * 8
                for j in range(8):
                    r = q0 + j
                    bc = plsc.load_gather(ew, [jnp.full((16,), r, _i32)])
                    for f in range(nf):
                        b[r, pl.ds(f * 16, 16)] = b[r, pl.ds(f * 16, 16)] * bc
                return carry
            lax.fori_loop(0, CH // 8, grp, 0)

        idx_start(0, 0)
        idx_wait(0, 0)
        g_start(0)
        idx_start(1, 1)

        def body(c, carry):
            for sl in range(3):
                cc = c * 3 + sl
                ns = (sl + 1) % 3
                nns = (sl + 2) % 3
                g_wait(sl)
                scale(sl)
                s_start(sl)

                @pl.when(cc + 2 < NCHUNK)
                def _():
                    @pl.when(cc >= 1)
                    def _():
                        s_wait(nns)
                    idx_start(cc + 2, nns)

                @pl.when(cc + 1 < NCHUNK)
                def _():
                    idx_wait(cc + 1, ns)
                    g_start(ns)
            return carry
        lax.fori_loop(0, NCHUNK // 3, body, 0)
        s_wait((NCHUNK - 3) % 3)
        s_wait((NCHUNK - 2) % 3)
        s_wait((NCHUNK - 1) % 3)
        plsc.subcore_barrier()
        pltpu.sync_copy(acc.at[pl.ds(si * RPT, RPT)], out_hbm.at[half, ci, si])


BM = 2048


def _mxu(a, b):
    return jnp.dot(a.astype(jnp.bfloat16), b.astype(jnp.bfloat16),
                   preferred_element_type=_f32)


def _relu(x):
    return jnp.maximum(x, 0.0)


def _dis_e(deg):
    return jnp.where(deg > 0.0, lax.rsqrt(jnp.maximum(deg, 1e-30)), 0.0)


def _split(q, qa, qb):
    qa[...] = q[:, :FH]
    qb[...] = q[:, FH:]


def _tc_in_body(xp, winp, binp, wc0, degp, qa, qb, sf):
    deg = jnp.sum(degp[...], axis=0)
    dis0 = lax.rsqrt(deg + 1.0)
    h = _mxu(xp[...], winp[...]) + binp[...]
    hw = _mxu(h, wc0[...])
    _split(hw * dis0[:, None], qa, qb)
    sf[...] = hw * (dis0 * dis0)[:, None]


_tc_in = pl.pallas_call(
    _tc_in_body,
    grid=(NP // BM,),
    in_specs=[
        pl.BlockSpec((BM, F), lambda i: (i, 0)),
        pl.BlockSpec((F, F), lambda i: (0, 0)),
        pl.BlockSpec((1, F), lambda i: (0, 0)),
        pl.BlockSpec((F, F), lambda i: (0, 0)),
        pl.BlockSpec((NW, BM), lambda i: (0, i)),
    ],
    out_specs=[
        pl.BlockSpec((BM, FH), lambda i: (i, 0)),
        pl.BlockSpec((BM, FH), lambda i: (i, 0)),
        pl.BlockSpec((BM, F), lambda i: (i, 0)),
    ],
    out_shape=[
        jax.ShapeDtypeStruct((NP, FH), _f32),
        jax.ShapeDtypeStruct((NP, FH), _f32),
        jax.ShapeDtypeStruct((NP, F), _f32),
    ],
)


def _acc_sum(accl, accr):
    return jnp.concatenate([accl[0] + accl[1], accr[0] + accr[1]], axis=-1)


def _tc_mid1_body(accl, accr, degp, sf, b0, wc, qa, qb):
    deg = jnp.sum(degp[...], axis=0)
    dis0 = lax.rsqrt(deg + 1.0)
    dise = _dis_e(deg)
    acs = _acc_sum(accl, accr)
    h = _relu(dis0[:, None] * acs + sf[...] + b0[...])
    q = _mxu(h, wc[...]) * dise[:, None]
    _split(q, qa, qb)


def _tc_mid_body(accl, accr, degp, bprev, wc, qa, qb):
    deg = jnp.sum(degp[...], axis=0)
    dise = _dis_e(deg)
    acs = _acc_sum(accl, accr)
    h = _relu(dise[:, None] * acs + bprev[...])
    q = _mxu(h, wc[...]) * dise[:, None]
    _split(q, qa, qb)


_ACC_SPEC = pl.BlockSpec((NC2, BM, FH), lambda i: (0, i, 0))
_QOUT = dict(
    out_specs=[
        pl.BlockSpec((BM, FH), lambda i: (i, 0)),
        pl.BlockSpec((BM, FH), lambda i: (i, 0)),
    ],
    out_shape=[
        jax.ShapeDtypeStruct((NP, FH), _f32),
        jax.ShapeDtypeStruct((NP, FH), _f32),
    ],
)

_tc_mid1 = pl.pallas_call(
    _tc_mid1_body,
    grid=(NP // BM,),
    in_specs=[
        _ACC_SPEC,
        _ACC_SPEC,
        pl.BlockSpec((NW, BM), lambda i: (0, i)),
        pl.BlockSpec((BM, F), lambda i: (i, 0)),
        pl.BlockSpec((1, F), lambda i: (0, 0)),
        pl.BlockSpec((F, F), lambda i: (0, 0)),
    ],
    **_QOUT,
)

_tc_mid = pl.pallas_call(
    _tc_mid_body,
    grid=(NP // BM,),
    in_specs=[
        _ACC_SPEC,
        _ACC_SPEC,
        pl.BlockSpec((NW, BM), lambda i: (0, i)),
        pl.BlockSpec((1, F), lambda i: (0, 0)),
        pl.BlockSpec((F, F), lambda i: (0, 0)),
    ],
    **_QOUT,
)


def _bn(x, g, b, eps=1e-5):
    m = jnp.mean(x, axis=0, keepdims=True)
    v = jnp.mean((x - m) * (x - m), axis=0, keepdims=True)
    return (x - m) / jnp.sqrt(v + eps) * g + b


def _tc_head_body(accl, accr, degp, oh, b3, g1, be1, w1, b1, g2, be2, w2, b2,
                  g3, be3, w3, b3h, out):
    deg = jnp.sum(degp[...], axis=0)
    dise = _dis_e(deg)
    acs = jnp.concatenate([accl[0] + accl[1], accr[0] + accr[1]], axis=-1)
    h4 = dise[:, None] * acs + b3[...]
    ohv = oh[...]
    psum = jax.lax.dot_general(ohv, h4, (((1,), (0,)), ((), ())),
                               precision=jax.lax.Precision.HIGHEST,
                               preferred_element_type=_f32)
    cnt = jnp.sum(ohv, axis=1, keepdims=True)
    pooled = psum / jnp.maximum(cnt, 1.0)
    o = _mxu(_relu(_bn(pooled, g1[...], be1[...])), w1[...]) + b1[...]
    o = _mxu(_relu(_bn(o, g2[...], be2[...])), w2[...]) + b2[...]
    o = _mxu(_relu(_bn(o, g3[...], be3[...])), w3[...]) + b3h[...]
    out[...] = o


_tc_head = pl.pallas_call(
    _tc_head_body,
    out_shape=jax.ShapeDtypeStruct((G, 128), _f32),
)


def kernel(x, pos, edge_index, edge_weight, batch, W_in, b_in,
           Wc0, bc0, Wc1, bc1, Wc2, bc2, Wc3, bc3,
           g1, be1, W1, b1, g2, be2, W2, b2, g3, be3, W3, b3):
    row = edge_index[0]
    col = edge_index[1]
    zpad_i = jnp.zeros((EP - E,), _i32)
    zpad_f = jnp.zeros((EP - E,), _f32)
    row_p = jnp.concatenate([row, zpad_i])
    col_p = jnp.concatenate([col, zpad_i])
    ew_p = jnp.concatenate([edge_weight, zpad_f])

    degp = _sc_deg(col_p, ew_p).reshape(NW, NP)

    xp = jnp.zeros((NP, F), _f32)
    xp = xp.at[:N, :DF].set(x).at[:N, DF:DF + PD].set(pos)
    winp = jnp.zeros((F, F), _f32).at[:DF + PD, :H].set(W_in)
    binp = jnp.zeros((1, F), _f32).at[0, :H].set(b_in)

    def padw(w):
        return jnp.zeros((F, F), _f32).at[:H, :H].set(w)

    def padb(b):
        return jnp.zeros((1, F), _f32).at[0, :H].set(b)

    wc = [padw(Wc0), padw(Wc1), padw(Wc2), padw(Wc3)]
    bc = [padb(bc0), padb(bc1), padb(bc2), padb(bc3)]

    def edge_pass(qa, qb):
        out = _sc_edge(qa, qb, row_p, col_p, ew_p)
        accl = out[0].reshape(NC2, NP, FH)
        accr = out[1].reshape(NC2, NP, FH)
        return accl, accr

    q0a, q0b, sf = _tc_in(xp, winp, binp, wc[0], degp)
    accl, accr = edge_pass(q0a, q0b)
    q1a, q1b = _tc_mid1(accl, accr, degp, sf, bc[0], wc[1])
    accl, accr = edge_pass(q1a, q1b)
    q2a, q2b = _tc_mid(accl, accr, degp, bc[1], wc[2])
    accl, accr = edge_pass(q2a, q2b)
    q3a, q3b = _tc_mid(accl, accr, degp, bc[2], wc[3])
    accl, accr = edge_pass(q3a, q3b)

    batch_p = jnp.concatenate([batch, jnp.full((NP - N,), 127, batch.dtype)])
    oh = (batch_p[None, :] == jnp.arange(G, dtype=batch.dtype)[:, None])
    oh = oh.astype(_f32)

    H2, H4 = H // 2, H // 4
    g1p = jnp.zeros((1, F), _f32).at[0, :H].set(g1)
    be1p = jnp.zeros((1, F), _f32).at[0, :H].set(be1)
    w1p = jnp.zeros((F, 80), _f32).at[:H, :H2].set(W1)
    b1p = jnp.zeros((1, 80), _f32).at[0, :H2].set(b1)
    g2p = jnp.zeros((1, 80), _f32).at[0, :H2].set(g2)
    be2p = jnp.zeros((1, 80), _f32).at[0, :H2].set(be2)
    w2p = jnp.zeros((80, 48), _f32).at[:H2, :H4].set(W2)
    b2p = jnp.zeros((1, 48), _f32).at[0, :H4].set(b2)
    g3p = jnp.zeros((1, 48), _f32).at[0, :H4].set(g3)
    be3p = jnp.zeros((1, 48), _f32).at[0, :H4].set(be3)
    w3p = jnp.zeros((48, 128), _f32).at[:H4, :NCLS].set(W3)
    b3p = jnp.zeros((1, 128), _f32).at[0, :NCLS].set(b3)

    o = _tc_head(accl, accr, degp, oh, bc[3], g1p, be1p, w1p, b1p,
                 g2p, be2p, w2p, b2p, g3p, be3p, w3p, b3p)
    return o[:, :NCLS]
